# baseline (speedup 1.0000x reference)
import math

import jax
import jax.numpy as jnp
from jax.experimental import pallas as pl
from jax.experimental.pallas import tpu as pltpu

ALPHA = 1e-4
BETA = 1.0
MAX_STD = 1000.0
MIN_STD = 1.0

_INV_SQRT_2PI = 1.0 / math.sqrt(2.0 * math.pi)


_HALF = 524288


def _fused_body(logits_ref, labels_ref, aw_ref, len_ref, params_hbm,
                loss_ref, nll_ref, buf0, buf1, sem0, sem1):
    bufs = (buf0, buf1)
    sems = (sem0, sem1)

    def copy(c):
        return pltpu.make_async_copy(
            params_hbm.at[pl.ds(c * _HALF, _HALF)], bufs[c], sems[c])

    copy(0).start()
    copy(1).start()

    lt = logits_ref[...]
    b = lt.shape[1]
    l0 = lt[0:1, :]
    l1 = lt[1:2, :]
    m = jnp.maximum(l0, l1)
    lse = m + jnp.log(jnp.exp(l0 - m) + jnp.exp(l1 - m))
    lab_row = labels_ref[...].reshape(1, b)
    picked = jnp.where(lab_row == 1, l1, l0) - lse
    nll = -jnp.sum(picked) / b
    labels = labels_ref[...].reshape(b, 1)

    aw = aw_ref[...].reshape(b, -1)
    seg_len = aw.shape[1]
    idx = jax.lax.broadcasted_iota(jnp.int32, (b, seg_len), 1)
    x = (idx.astype(jnp.float32) + 1.0) / seg_len
    sums = jnp.sum(aw, axis=1, keepdims=True)
    means = jnp.sum(x * aw, axis=1, keepdims=True) / sums
    len_f = len_ref[...].reshape(b, 1).astype(jnp.float32)
    ideal_stds = jnp.where(labels == 1, MIN_STD / len_f, MAX_STD / len_f)
    z = (x - means) / ideal_stds
    r_hats = jnp.exp(-0.5 * z * z) * (_INV_SQRT_2PI / ideal_stds)
    rs = r_hats / (jnp.sum(r_hats, axis=1, keepdims=True) + 1e-6)
    diff = aw - rs
    aw_penalty = (BETA / 2.0) * jnp.mean(diff * diff)

    acc = jnp.float32(0.0)
    for c in range(2):
        copy(c).wait()
        acc = acc + jnp.sum(jnp.abs(bufs[c][...].reshape(-1, 512)))
    penalty = (ALPHA / 2.0) * acc

    nll_ref[...] = nll
    loss_ref[...] = nll + penalty + aw_penalty


@jax.jit
def _run(logits, labels, attention_weights, lengths, params):
    vmem = pl.BlockSpec(memory_space=pltpu.VMEM)
    smem = pl.BlockSpec(memory_space=pltpu.SMEM)
    out = pl.pallas_call(
        _fused_body,
        in_specs=[vmem, vmem, vmem, vmem,
                  pl.BlockSpec(memory_space=pltpu.HBM)],
        out_specs=(smem, smem),
        out_shape=(
            jax.ShapeDtypeStruct((), jnp.float32),
            jax.ShapeDtypeStruct((), jnp.float32),
        ),
        scratch_shapes=[
            pltpu.VMEM((_HALF,), jnp.float32),
            pltpu.VMEM((_HALF,), jnp.float32),
            pltpu.SemaphoreType.DMA,
            pltpu.SemaphoreType.DMA,
        ],
    )(logits.T, labels, attention_weights, lengths, params)
    return out


def kernel(logits, labels, attention_weights, lengths, params):
    return _run(logits, labels, attention_weights, lengths, params)


# speedup vs baseline: 1.0154x; 1.0154x over previous
import math

import jax
import jax.numpy as jnp
from jax.experimental import pallas as pl
from jax.experimental.pallas import tpu as pltpu

ALPHA = 1e-4
BETA = 1.0
MAX_STD = 1000.0
MIN_STD = 1.0

_INV_SQRT_2PI = 1.0 / math.sqrt(2.0 * math.pi)


def _fused_body(logits_ref, labels_ref, aw_ref, len_ref, params_ref,
                loss_ref, nll_ref):
    lt = logits_ref[...]
    b = lt.shape[1]
    l0 = lt[0:1, :]
    l1 = lt[1:2, :]
    m = jnp.maximum(l0, l1)
    lse = m + jnp.log(jnp.exp(l0 - m) + jnp.exp(l1 - m))
    lab_row = labels_ref[...].reshape(1, b)
    picked = jnp.where(lab_row == 1, l1, l0) - lse
    nll = -jnp.sum(picked) / b
    labels = labels_ref[...].reshape(b, 1)

    aw = aw_ref[...].reshape(b, -1)
    seg_len = aw.shape[1]
    idx = jax.lax.broadcasted_iota(jnp.int32, (b, seg_len), 1)
    x = (idx.astype(jnp.float32) + 1.0) / seg_len
    sums = jnp.sum(aw, axis=1, keepdims=True)
    means = jnp.sum(x * aw, axis=1, keepdims=True) / sums
    len_f = len_ref[...].reshape(b, 1).astype(jnp.float32)
    ideal_stds = jnp.where(labels == 1, MIN_STD / len_f, MAX_STD / len_f)
    z = (x - means) / ideal_stds
    r_hats = jnp.exp(-0.5 * z * z) * (_INV_SQRT_2PI / ideal_stds)
    rs = r_hats / (jnp.sum(r_hats, axis=1, keepdims=True) + 1e-6)
    diff = aw - rs
    aw_penalty = (BETA / 2.0) * jnp.mean(diff * diff)

    p = params_ref[...].reshape(-1, 512)
    penalty = (ALPHA / 2.0) * jnp.sum(jnp.abs(p))

    nll_ref[...] = nll
    loss_ref[...] = nll + penalty + aw_penalty


@jax.jit
def _run(logits, labels, attention_weights, lengths, params):
    vmem = pl.BlockSpec(memory_space=pltpu.VMEM)
    smem = pl.BlockSpec(memory_space=pltpu.SMEM)
    out = pl.pallas_call(
        _fused_body,
        in_specs=[vmem] * 5,
        out_specs=(smem, smem),
        out_shape=(
            jax.ShapeDtypeStruct((), jnp.float32),
            jax.ShapeDtypeStruct((), jnp.float32),
        ),
    )(logits.T, labels, attention_weights, lengths, params)
    return out


def kernel(logits, labels, attention_weights, lengths, params):
    return _run(logits, labels, attention_weights, lengths, params)
